# baseline (speedup 1.0000x reference)
import functools

import jax
import jax.numpy as jnp
from jax import lax
from jax.experimental import pallas as pl
from jax.experimental.pallas import tpu as pltpu
from jax.experimental.pallas import tpu_sc as plsc

NUM_SAMPLED = 255
S_PAD = 256


@functools.lru_cache(maxsize=None)
def _make_sc_gather(V, D, B):
    info = plsc.get_sparse_core_info()
    NC, NS = info.num_cores, info.num_subcores
    NW = NC * NS
    assert B % NW == 0
    b_per_w = B // NW
    assert b_per_w % 8 == 0
    chunks = []
    off = 0
    while off < b_per_w:
        c = min(128, b_per_w - off)
        chunks.append((off, c))
        off += c

    mesh = plsc.VectorSubcoreMesh(core_axis_name="c", subcore_axis_name="s")

    @functools.partial(
        pl.kernel,
        mesh=mesh,
        out_type=jax.ShapeDtypeStruct((B, D), jnp.float32),
        scratch_types=[
            pltpu.VMEM((b_per_w,), jnp.int32),
            pltpu.VMEM((b_per_w, D), jnp.float32),
            pltpu.SemaphoreType.DMA,
        ],
        compiler_params=pltpu.CompilerParams(use_tc_tiling_on_sc=False),
    )
    def gather(table_hbm, idx_hbm, out_hbm, idx_v, rows_v, sem):
        wid = lax.axis_index("s") * NC + lax.axis_index("c")
        base = wid * b_per_w
        pltpu.sync_copy(idx_hbm.at[pl.ds(base, b_per_w)], idx_v)
        copies = [
            pltpu.async_copy(
                table_hbm.at[idx_v.at[pl.ds(o, c)]], rows_v.at[pl.ds(o, c)], sem
            )
            for (o, c) in chunks
        ]
        for cp in copies:
            cp.wait()
        pltpu.sync_copy(rows_v, out_hbm.at[pl.ds(base, b_per_w)])

    return gather


def _dense_body(logv1_ref, idx_ref, uv_ref, tw_ref, sw_ref, cmp_ref, nlse_ref,
                out_ref):
    logv1 = logv1_ref[0]
    uv = uv_ref[...]
    c = idx_ref[...].astype(jnp.float32)
    p_true = (jnp.log(c + 2.0) - jnp.log(c + 1.0)) / logv1
    lte = jnp.log(1.0 - jnp.exp(NUM_SAMPLED * jnp.log(1.0 - p_true)))
    true_logit = jnp.sum(uv * tw_ref[...], axis=1, keepdims=True) - lte

    sl = lax.dot_general(
        uv, sw_ref[...], (((1,), (1,)), ((), ())),
        preferred_element_type=jnp.float32,
    )
    sl = sl + nlse_ref[...]
    acc = (cmp_ref[...] == idx_ref[...]).astype(jnp.float32)
    sl = sl - acc * 1e9

    m = jnp.maximum(jnp.max(sl, axis=1, keepdims=True), true_logit)
    z = jnp.sum(jnp.exp(sl - m), axis=1, keepdims=True) + jnp.exp(true_logit - m)
    out_ref[...] = jnp.log(z) + m - true_logit


def _dense(logv1, idx2d, uv, tw, sw, cmp2d, nlse2d, *, interpret=False):
    B, D = uv.shape
    BB = min(B, 1024)
    grid = (B // BB,)
    return pl.pallas_call(
        _dense_body,
        grid=grid,
        in_specs=[
            pl.BlockSpec(memory_space=pltpu.SMEM),
            pl.BlockSpec((BB, 1), lambda i: (i, 0)),
            pl.BlockSpec((BB, D), lambda i: (i, 0)),
            pl.BlockSpec((BB, D), lambda i: (i, 0)),
            pl.BlockSpec((S_PAD, D), lambda i: (0, 0)),
            pl.BlockSpec((1, S_PAD), lambda i: (0, 0)),
            pl.BlockSpec((1, S_PAD), lambda i: (0, 0)),
        ],
        out_specs=pl.BlockSpec((BB, 1), lambda i: (i, 0)),
        out_shape=jax.ShapeDtypeStruct((B, 1), jnp.float32),
        interpret=interpret,
    )(logv1, idx2d, uv, tw, sw, cmp2d, nlse2d)


def kernel(item_embeddings, user_vec, item_idx, zero_bias):
    V, D = item_embeddings.shape
    B = user_vec.shape[0]
    idx = item_idx.reshape(-1).astype(jnp.int32)

    logv1 = jnp.log(jnp.float32(V) + 1.0)
    skey = jax.random.fold_in(jax.random.key(0), 12345)
    u = jax.random.uniform(skey, (NUM_SAMPLED,), dtype=jnp.float32)
    s = jnp.floor(jnp.exp(u * logv1)) - 1.0
    sampled = jnp.clip(s, 0, V - 1).astype(jnp.int32)
    cs = sampled.astype(jnp.float32)
    p_samp = (jnp.log(cs + 2.0) - jnp.log(cs + 1.0)) / logv1
    nlse = -jnp.log(-jnp.expm1(NUM_SAMPLED * jnp.log1p(-p_samp)))
    nlse_pad = jnp.concatenate([nlse, jnp.full((1,), -1e30, jnp.float32)])
    cmp_pad = jnp.concatenate([sampled, jnp.full((1,), -1, jnp.int32)])
    gidx_pad = jnp.concatenate([sampled, jnp.zeros((1,), jnp.int32)])

    all_idx = jnp.concatenate([idx, gidx_pad])
    gathered = _make_sc_gather(V, D, B + S_PAD)(item_embeddings, all_idx)
    tw = gathered[:B]
    sw = gathered[B:]

    loss = _dense(
        logv1.reshape(1),
        idx.reshape(B, 1),
        user_vec,
        tw,
        sw,
        cmp_pad.reshape(1, S_PAD),
        nlse_pad.reshape(1, S_PAD),
    )
    return loss
